# baseline (speedup 1.0000x reference)
import functools

import jax
import jax.numpy as jnp
from jax import lax
from jax.experimental import pallas as pl
from jax.experimental.pallas import tpu as pltpu
from jax.experimental.pallas import tpu_sc as plsc

N_NODES = 10000
N_PAD = 10240
N_EDGES = 320000
IN_CH = 128
HID = 256
OUT_CH = 128

NC = 2
NS = 16
NW = NC * NS
CHUNK = 128
NCHUNK = 80
EW = NCHUNK * CHUNK
E_PAD = NW * EW
TRASH = N_PAD - 1

ROWS_PER_TILE = N_PAD // NS

_MESH = plsc.VectorSubcoreMesh(core_axis_name="c", subcore_axis_name="s")


@functools.partial(
    pl.kernel,
    out_type=[
        jax.ShapeDtypeStruct((N_PAD,), jnp.float32),
        jax.ShapeDtypeStruct((N_PAD,), jnp.float32),
    ],
    mesh=_MESH,
    scratch_types=[
        pltpu.VMEM((CHUNK,), jnp.int32),
        pltpu.VMEM((CHUNK,), jnp.int32),
        pltpu.VMEM((CHUNK,), jnp.float32),
        pltpu.VMEM_SHARED((N_PAD,), jnp.float32),
        pltpu.SemaphoreType.DMA,
        pltpu.SemaphoreType.DMA,
    ],
)
def _sc_degree(ei_hbm, zeros1_hbm, dp0_hbm, dp1_hbm, dst0_v, dst1_v, ones_v,
               deg_sh, isem0, isem1):
    c = lax.axis_index("c")
    s = lax.axis_index("s")
    w = c * NS + s
    r0 = s * ROWS_PER_TILE
    e0 = w * EW
    pltpu.sync_copy(zeros1_hbm.at[pl.ds(r0, ROWS_PER_TILE)],
                    deg_sh.at[pl.ds(r0, ROWS_PER_TILE)])
    for k in range(CHUNK // 16):
        ones_v[pl.ds(16 * k, 16)] = jnp.ones((16,), jnp.float32)
    dst = (dst0_v, dst1_v)
    isems = (isem0, isem1)
    pltpu.async_copy(ei_hbm.at[1, pl.ds(e0, CHUNK)], dst[0], isems[0])
    pltpu.async_copy(ei_hbm.at[1, pl.ds(e0 + CHUNK, CHUNK)], dst[1], isems[1])
    plsc.subcore_barrier()

    @pl.loop(0, NCHUNK, step=2)
    def _loop(j):
        for b in range(2):
            i = j + b
            pltpu.make_async_copy(ei_hbm.at[1, pl.ds(e0 + i * CHUNK, CHUNK)],
                                  dst[b], isems[b]).wait()
            pltpu.sync_copy(ones_v, deg_sh.at[dst[b]], add=True)

            @pl.when(i + 2 < NCHUNK)
            def _():
                pltpu.async_copy(
                    ei_hbm.at[1, pl.ds(e0 + (i + 2) * CHUNK, CHUNK)],
                    dst[b], isems[b])

    plsc.subcore_barrier()

    @pl.when(c == 0)
    def _():
        pltpu.sync_copy(deg_sh.at[pl.ds(r0, ROWS_PER_TILE)],
                        dp0_hbm.at[pl.ds(r0, ROWS_PER_TILE)])

    @pl.when(c == 1)
    def _():
        pltpu.sync_copy(deg_sh.at[pl.ds(r0, ROWS_PER_TILE)],
                        dp1_hbm.at[pl.ds(r0, ROWS_PER_TILE)])


@functools.partial(
    pl.kernel,
    out_type=[
        jax.ShapeDtypeStruct((N_PAD, IN_CH), jnp.float32),
        jax.ShapeDtypeStruct((N_PAD, IN_CH), jnp.float32),
    ],
    mesh=_MESH,
    scratch_types=[
        pltpu.VMEM((CHUNK,), jnp.int32),
        pltpu.VMEM((CHUNK,), jnp.int32),
        pltpu.VMEM((CHUNK,), jnp.int32),
        pltpu.VMEM((CHUNK,), jnp.int32),
        pltpu.VMEM((CHUNK, IN_CH), jnp.float32),
        pltpu.VMEM((CHUNK, IN_CH), jnp.float32),
        pltpu.VMEM_SHARED((N_PAD, IN_CH), jnp.float32),
        pltpu.SemaphoreType.DMA,
        pltpu.SemaphoreType.DMA,
        pltpu.SemaphoreType.DMA,
        pltpu.SemaphoreType.DMA,
        pltpu.SemaphoreType.DMA,
        pltpu.SemaphoreType.DMA,
    ],
)
def _sc_aggregate(y_hbm, ei_hbm, zeros_hbm, z0_hbm, z1_hbm,
                  src0_v, src1_v, dst0_v, dst1_v, rows0_v, rows1_v, z_sh,
                  gsem0, gsem1, ssem0, ssem1, dsem0, dsem1):
    c = lax.axis_index("c")
    s = lax.axis_index("s")
    w = c * NS + s
    r0 = s * ROWS_PER_TILE
    e0 = w * EW
    pltpu.sync_copy(zeros_hbm.at[pl.ds(r0, ROWS_PER_TILE)],
                    z_sh.at[pl.ds(r0, ROWS_PER_TILE)])
    src = (src0_v, src1_v)
    dst = (dst0_v, dst1_v)
    rows = (rows0_v, rows1_v)
    gsems = (gsem0, gsem1)
    ssems = (ssem0, ssem1)
    dsems = (dsem0, dsem1)
    pltpu.sync_copy(ei_hbm.at[0, pl.ds(e0, CHUNK)], src[0])
    pltpu.sync_copy(ei_hbm.at[1, pl.ds(e0, CHUNK)], dst[0])
    pltpu.async_copy(ei_hbm.at[0, pl.ds(e0 + CHUNK, CHUNK)], src[1], ssems[1])
    pltpu.async_copy(ei_hbm.at[1, pl.ds(e0 + CHUNK, CHUNK)], dst[1], dsems[1])
    plsc.subcore_barrier()
    pltpu.async_copy(y_hbm.at[src[0]], rows[0], gsems[0])

    @pl.loop(0, NCHUNK, step=2)
    def _loop(j):
        for b in range(2):
            i = j + b
            nb = 1 - b
            pltpu.make_async_copy(y_hbm.at[src[b]], rows[b], gsems[b]).wait()

            @pl.when(i + 1 < NCHUNK)
            def _():
                pltpu.make_async_copy(
                    ei_hbm.at[0, pl.ds(e0 + (i + 1) * CHUNK, CHUNK)],
                    src[nb], ssems[nb]).wait()
                pltpu.make_async_copy(
                    ei_hbm.at[1, pl.ds(e0 + (i + 1) * CHUNK, CHUNK)],
                    dst[nb], dsems[nb]).wait()
                pltpu.async_copy(y_hbm.at[src[nb]], rows[nb], gsems[nb])

            pltpu.sync_copy(rows[b], z_sh.at[dst[b]], add=True)

            @pl.when(i + 2 < NCHUNK)
            def _():
                pltpu.async_copy(
                    ei_hbm.at[0, pl.ds(e0 + (i + 2) * CHUNK, CHUNK)],
                    src[b], ssems[b])
                pltpu.async_copy(
                    ei_hbm.at[1, pl.ds(e0 + (i + 2) * CHUNK, CHUNK)],
                    dst[b], dsems[b])

    plsc.subcore_barrier()

    @pl.when(c == 0)
    def _():
        pltpu.sync_copy(z_sh.at[pl.ds(r0, ROWS_PER_TILE)],
                        z0_hbm.at[pl.ds(r0, ROWS_PER_TILE)])

    @pl.when(c == 1)
    def _():
        pltpu.sync_copy(z_sh.at[pl.ds(r0, ROWS_PER_TILE)],
                        z1_hbm.at[pl.ds(r0, ROWS_PER_TILE)])


_BLK = 1000
_GRID = N_NODES // _BLK


def _t0_body(dp0_ref, dp1_ref, x_ref, d_ref, y_ref):
    deg = dp0_ref[...] + dp1_ref[...] + 1.0
    d = lax.rsqrt(deg)
    d_ref[...] = d
    y_ref[...] = x_ref[...] * d


def _tc_scale(dp0, dp1, x):
    row = lambda i: (i, 0)
    return pl.pallas_call(
        _t0_body,
        grid=(_GRID,),
        in_specs=[
            pl.BlockSpec((_BLK, 1), row),
            pl.BlockSpec((_BLK, 1), row),
            pl.BlockSpec((_BLK, IN_CH), row),
        ],
        out_specs=[
            pl.BlockSpec((_BLK, 1), row),
            pl.BlockSpec((_BLK, IN_CH), row),
        ],
        out_shape=[
            jax.ShapeDtypeStruct((N_NODES, 1), jnp.float32),
            jax.ShapeDtypeStruct((N_NODES, IN_CH), jnp.float32),
        ],
    )(dp0.reshape(N_PAD, 1), dp1.reshape(N_PAD, 1), x)


def _t1_body(za_ref, zb_ref, y1_ref, d_ref, w1_ref, b1_ref, w2_ref, y2_ref):
    d = d_ref[...]
    m = (za_ref[...] + zb_ref[...] + y1_ref[...]) * d
    h = jnp.dot(m, w1_ref[...], preferred_element_type=jnp.float32)
    h = jnp.maximum(h + b1_ref[...], 0.0)
    g = jnp.dot(h, w2_ref[...], preferred_element_type=jnp.float32)
    y2_ref[...] = g * d


def _tc_mid(za, zb, y1, d, W1, b1, W2):
    row = lambda i: (i, 0)
    full = lambda i: (0, 0)
    return pl.pallas_call(
        _t1_body,
        grid=(_GRID,),
        in_specs=[
            pl.BlockSpec((_BLK, IN_CH), row),
            pl.BlockSpec((_BLK, IN_CH), row),
            pl.BlockSpec((_BLK, IN_CH), row),
            pl.BlockSpec((_BLK, 1), row),
            pl.BlockSpec((IN_CH, HID), full),
            pl.BlockSpec((1, HID), full),
            pl.BlockSpec((HID, OUT_CH), full),
        ],
        out_specs=pl.BlockSpec((_BLK, OUT_CH), row),
        out_shape=jax.ShapeDtypeStruct((N_NODES, OUT_CH), jnp.float32),
    )(za, zb, y1, d, W1, b1.reshape(1, HID), W2)


def _t2_body(za_ref, zb_ref, y2_ref, d_ref, b2_ref, out_ref):
    u = (za_ref[...] + zb_ref[...] + y2_ref[...]) * d_ref[...]
    out_ref[...] = u + b2_ref[...]


def _tc_final(za, zb, y2, d, b2):
    row = lambda i: (i, 0)
    full = lambda i: (0, 0)
    return pl.pallas_call(
        _t2_body,
        grid=(_GRID,),
        in_specs=[
            pl.BlockSpec((_BLK, OUT_CH), row),
            pl.BlockSpec((_BLK, OUT_CH), row),
            pl.BlockSpec((_BLK, OUT_CH), row),
            pl.BlockSpec((_BLK, 1), row),
            pl.BlockSpec((1, OUT_CH), full),
        ],
        out_specs=pl.BlockSpec((_BLK, OUT_CH), row),
        out_shape=jax.ShapeDtypeStruct((N_NODES, OUT_CH), jnp.float32),
    )(za, zb, y2, d, b2.reshape(1, OUT_CH))


@jax.jit
def kernel(x, edge_index, W1, b1, W2, b2):
    ei = edge_index.astype(jnp.int32)
    n_dummy = E_PAD - N_EDGES
    pad = jnp.stack([
        jnp.zeros((n_dummy,), jnp.int32),
        jnp.full((n_dummy,), TRASH, jnp.int32),
    ])
    ei = jnp.concatenate([ei, pad], axis=1)
    zeros1 = jnp.zeros((N_PAD,), jnp.float32)
    zeros128 = jnp.zeros((N_PAD, IN_CH), jnp.float32)

    dp0, dp1 = _sc_degree(ei, zeros1)
    d, y1 = _tc_scale(dp0, dp1, x)
    z1a, z1b = _sc_aggregate(y1, ei, zeros128)
    y2 = _tc_mid(z1a, z1b, y1, d, W1, b1, W2)
    z2a, z2b = _sc_aggregate(y2, ei, zeros128)
    return _tc_final(z2a, z2b, y2, d, b2)


# speedup vs baseline: 3.3686x; 3.3686x over previous
import functools

import jax
import jax.numpy as jnp
from jax import lax
from jax.experimental import pallas as pl
from jax.experimental.pallas import tpu as pltpu
from jax.experimental.pallas import tpu_sc as plsc

N_NODES = 10000
N_PAD = 10240
N_EDGES = 320000
IN_CH = 128
HID = 256
OUT_CH = 128

NC = 2
NS = 16
NW = NC * NS
CHUNK = 128
NCHUNK = 80
EW = NCHUNK * CHUNK
E_PAD = NW * EW
TRASH = N_PAD - 1

ROWS_PER_TILE = N_PAD // NS

_MESH = plsc.VectorSubcoreMesh(core_axis_name="c", subcore_axis_name="s")


@functools.partial(
    pl.kernel,
    out_type=[
        jax.ShapeDtypeStruct((N_PAD,), jnp.float32),
        jax.ShapeDtypeStruct((N_PAD,), jnp.float32),
    ],
    mesh=_MESH,
    scratch_types=[
        pltpu.VMEM((NCHUNK, CHUNK), jnp.int32),
        pltpu.VMEM((CHUNK,), jnp.float32),
        pltpu.VMEM_SHARED((N_PAD,), jnp.float32),
        pltpu.SemaphoreType.DMA,
    ],
)
def _sc_degree(dst2_hbm, zeros1_hbm, dp0_hbm, dp1_hbm, dst_v, ones_v,
               deg_sh, isem):
    c = lax.axis_index("c")
    s = lax.axis_index("s")
    w = c * NS + s
    r0 = s * ROWS_PER_TILE
    pltpu.sync_copy(zeros1_hbm.at[pl.ds(r0, ROWS_PER_TILE)],
                    deg_sh.at[pl.ds(r0, ROWS_PER_TILE)])
    for k in range(CHUNK // 16):
        ones_v[pl.ds(16 * k, 16)] = jnp.ones((16,), jnp.float32)
    pltpu.sync_copy(dst2_hbm.at[w], dst_v)
    plsc.subcore_barrier()

    @pl.loop(0, NCHUNK)
    def _loop(i):
        pltpu.sync_copy(ones_v, deg_sh.at[dst_v.at[i]], add=True)

    plsc.subcore_barrier()

    @pl.when(c == 0)
    def _():
        pltpu.sync_copy(deg_sh.at[pl.ds(r0, ROWS_PER_TILE)],
                        dp0_hbm.at[pl.ds(r0, ROWS_PER_TILE)])

    @pl.when(c == 1)
    def _():
        pltpu.sync_copy(deg_sh.at[pl.ds(r0, ROWS_PER_TILE)],
                        dp1_hbm.at[pl.ds(r0, ROWS_PER_TILE)])


@functools.partial(
    pl.kernel,
    out_type=[
        jax.ShapeDtypeStruct((N_PAD, IN_CH), jnp.float32),
        jax.ShapeDtypeStruct((N_PAD, IN_CH), jnp.float32),
    ],
    mesh=_MESH,
    scratch_types=[
        pltpu.VMEM((CHUNK,), jnp.int32),
        pltpu.VMEM((CHUNK,), jnp.int32),
        pltpu.VMEM((CHUNK,), jnp.int32),
        pltpu.VMEM((CHUNK,), jnp.int32),
        pltpu.VMEM((CHUNK, IN_CH), jnp.float32),
        pltpu.VMEM((CHUNK, IN_CH), jnp.float32),
        pltpu.VMEM_SHARED((N_PAD, IN_CH), jnp.float32),
        pltpu.SemaphoreType.DMA,
        pltpu.SemaphoreType.DMA,
        pltpu.SemaphoreType.DMA,
        pltpu.SemaphoreType.DMA,
        pltpu.SemaphoreType.DMA,
        pltpu.SemaphoreType.DMA,
    ],
)
def _sc_aggregate(y_hbm, ei_hbm, zeros_hbm, z0_hbm, z1_hbm,
                  src0_v, src1_v, dst0_v, dst1_v, rows0_v, rows1_v, z_sh,
                  gsem0, gsem1, ssem0, ssem1, dsem0, dsem1):
    c = lax.axis_index("c")
    s = lax.axis_index("s")
    w = c * NS + s
    r0 = s * ROWS_PER_TILE
    e0 = w * EW
    pltpu.sync_copy(zeros_hbm.at[pl.ds(r0, ROWS_PER_TILE)],
                    z_sh.at[pl.ds(r0, ROWS_PER_TILE)])
    src = (src0_v, src1_v)
    dst = (dst0_v, dst1_v)
    rows = (rows0_v, rows1_v)
    gsems = (gsem0, gsem1)
    ssems = (ssem0, ssem1)
    dsems = (dsem0, dsem1)
    pltpu.sync_copy(ei_hbm.at[0, pl.ds(e0, CHUNK)], src[0])
    pltpu.sync_copy(ei_hbm.at[1, pl.ds(e0, CHUNK)], dst[0])
    pltpu.async_copy(ei_hbm.at[0, pl.ds(e0 + CHUNK, CHUNK)], src[1], ssems[1])
    pltpu.async_copy(ei_hbm.at[1, pl.ds(e0 + CHUNK, CHUNK)], dst[1], dsems[1])
    plsc.subcore_barrier()
    pltpu.async_copy(y_hbm.at[src[0]], rows[0], gsems[0])

    @pl.loop(0, NCHUNK, step=2)
    def _loop(j):
        for b in range(2):
            i = j + b
            nb = 1 - b
            pltpu.make_async_copy(y_hbm.at[src[b]], rows[b], gsems[b]).wait()

            @pl.when(i + 1 < NCHUNK)
            def _():
                pltpu.make_async_copy(
                    ei_hbm.at[0, pl.ds(e0 + (i + 1) * CHUNK, CHUNK)],
                    src[nb], ssems[nb]).wait()
                pltpu.make_async_copy(
                    ei_hbm.at[1, pl.ds(e0 + (i + 1) * CHUNK, CHUNK)],
                    dst[nb], dsems[nb]).wait()
                pltpu.async_copy(y_hbm.at[src[nb]], rows[nb], gsems[nb])

            pltpu.sync_copy(rows[b], z_sh.at[dst[b]], add=True)

            @pl.when(i + 2 < NCHUNK)
            def _():
                pltpu.async_copy(
                    ei_hbm.at[0, pl.ds(e0 + (i + 2) * CHUNK, CHUNK)],
                    src[b], ssems[b])
                pltpu.async_copy(
                    ei_hbm.at[1, pl.ds(e0 + (i + 2) * CHUNK, CHUNK)],
                    dst[b], dsems[b])

    plsc.subcore_barrier()

    @pl.when(c == 0)
    def _():
        pltpu.sync_copy(z_sh.at[pl.ds(r0, ROWS_PER_TILE)],
                        z0_hbm.at[pl.ds(r0, ROWS_PER_TILE)])

    @pl.when(c == 1)
    def _():
        pltpu.sync_copy(z_sh.at[pl.ds(r0, ROWS_PER_TILE)],
                        z1_hbm.at[pl.ds(r0, ROWS_PER_TILE)])


_BLK = 1000
_GRID = N_NODES // _BLK


def _t0_body(dp0_ref, dp1_ref, x_ref, d_ref, y_ref):
    deg = dp0_ref[...] + dp1_ref[...] + 1.0
    d = lax.rsqrt(deg)
    d_ref[...] = d
    y_ref[...] = x_ref[...] * d


def _tc_scale(dp0, dp1, x):
    row = lambda i: (i, 0)
    return pl.pallas_call(
        _t0_body,
        grid=(_GRID,),
        in_specs=[
            pl.BlockSpec((_BLK, 1), row),
            pl.BlockSpec((_BLK, 1), row),
            pl.BlockSpec((_BLK, IN_CH), row),
        ],
        out_specs=[
            pl.BlockSpec((_BLK, 1), row),
            pl.BlockSpec((_BLK, IN_CH), row),
        ],
        out_shape=[
            jax.ShapeDtypeStruct((N_NODES, 1), jnp.float32),
            jax.ShapeDtypeStruct((N_NODES, IN_CH), jnp.float32),
        ],
    )(dp0.reshape(N_PAD, 1), dp1.reshape(N_PAD, 1), x)


def _t1_body(za_ref, zb_ref, y1_ref, d_ref, w1_ref, b1_ref, w2_ref, y2_ref):
    d = d_ref[...]
    m = (za_ref[...] + zb_ref[...] + y1_ref[...]) * d
    h = jnp.dot(m, w1_ref[...], preferred_element_type=jnp.float32)
    h = jnp.maximum(h + b1_ref[...], 0.0)
    g = jnp.dot(h, w2_ref[...], preferred_element_type=jnp.float32)
    y2_ref[...] = g * d


def _tc_mid(za, zb, y1, d, W1, b1, W2):
    row = lambda i: (i, 0)
    full = lambda i: (0, 0)
    return pl.pallas_call(
        _t1_body,
        grid=(_GRID,),
        in_specs=[
            pl.BlockSpec((_BLK, IN_CH), row),
            pl.BlockSpec((_BLK, IN_CH), row),
            pl.BlockSpec((_BLK, IN_CH), row),
            pl.BlockSpec((_BLK, 1), row),
            pl.BlockSpec((IN_CH, HID), full),
            pl.BlockSpec((1, HID), full),
            pl.BlockSpec((HID, OUT_CH), full),
        ],
        out_specs=pl.BlockSpec((_BLK, OUT_CH), row),
        out_shape=jax.ShapeDtypeStruct((N_NODES, OUT_CH), jnp.float32),
    )(za, zb, y1, d, W1, b1.reshape(1, HID), W2)


def _t2_body(za_ref, zb_ref, y2_ref, d_ref, b2_ref, out_ref):
    u = (za_ref[...] + zb_ref[...] + y2_ref[...]) * d_ref[...]
    out_ref[...] = u + b2_ref[...]


def _tc_final(za, zb, y2, d, b2):
    row = lambda i: (i, 0)
    full = lambda i: (0, 0)
    return pl.pallas_call(
        _t2_body,
        grid=(_GRID,),
        in_specs=[
            pl.BlockSpec((_BLK, OUT_CH), row),
            pl.BlockSpec((_BLK, OUT_CH), row),
            pl.BlockSpec((_BLK, OUT_CH), row),
            pl.BlockSpec((_BLK, 1), row),
            pl.BlockSpec((1, OUT_CH), full),
        ],
        out_specs=pl.BlockSpec((_BLK, OUT_CH), row),
        out_shape=jax.ShapeDtypeStruct((N_NODES, OUT_CH), jnp.float32),
    )(za, zb, y2, d, b2.reshape(1, OUT_CH))


@jax.jit
def kernel(x, edge_index, W1, b1, W2, b2):
    ei = edge_index.astype(jnp.int32)
    n_dummy = E_PAD - N_EDGES
    lin = jnp.arange(n_dummy, dtype=jnp.int32)
    pad = jnp.stack([lin % N_NODES, N_NODES + lin % (N_PAD - N_NODES)])
    ei = jnp.concatenate([ei, pad], axis=1)
    dst2 = ei[1].reshape(NW, NCHUNK, CHUNK)
    zeros1 = jnp.zeros((N_PAD,), jnp.float32)
    zeros128 = jnp.zeros((N_PAD, IN_CH), jnp.float32)

    dp0, dp1 = _sc_degree(dst2, zeros1)
    d, y1 = _tc_scale(dp0, dp1, x)
    z1a, z1b = _sc_aggregate(y1, ei, zeros128)
    y2 = _tc_mid(z1a, z1b, y1, d, W1, b1, W2)
    z2a, z2b = _sc_aggregate(y2, ei, zeros128)
    return _tc_final(z2a, z2b, y2, d, b2)
